# Initial kernel scaffold; baseline (speedup 1.0000x reference)
import functools

import jax
import jax.numpy as jnp
from jax import lax
from jax.experimental import pallas as pl
from jax.experimental.pallas import tpu as pltpu
from jax.experimental.pallas import tpu_sc as plsc

N = 10000
E = 320000
D_IN = 128
H1, C1 = 8, 16
OUT = 64

NC, NS = 2, 16
NW = NC * NS
R = 10240
RPT = R // NS
ETOT = E + N
CH = 120
EPW = 10320
EPAD = EPW * NW
NCHUNK = EPW // CH

TS1 = 144
TS2 = 80
TD = 16


def _mesh():
    return plsc.VectorSubcoreMesh(
        core_axis_name="c", subcore_axis_name="s", num_cores=NC, num_subcores=NS
    )


def _edge_body(nheads, ts, tabs_hbm, tabd_hbm, src_hbm, dst_hbm, zrow_hbm,
               out_hbm, sidx, didx, rows_s, rows_d, out_rows, wtmp, acc):
    cid = lax.axis_index("c")
    sid = lax.axis_index("s")
    wid = sid * NC + cid
    hb = nheads * 16

    pltpu.sync_copy(zrow_hbm, acc.at[pl.ds(sid * RPT, RPT)])
    plsc.subcore_barrier()

    @pl.loop(0, NCHUNK)
    def _chunks(c):
        off = wid * EPW + c * CH
        pltpu.sync_copy(src_hbm.at[pl.ds(off, CH)], sidx)
        pltpu.sync_copy(dst_hbm.at[pl.ds(off, CH)], didx)
        pltpu.sync_copy(tabs_hbm.at[sidx], rows_s)
        pltpu.sync_copy(tabd_hbm.at[didx], rows_d)

        @pl.loop(0, CH)
        def _edges(i):
            als = rows_s[i, pl.ds(hb, 16)]
            ald = rows_d[i, pl.ds(0, 16)]
            t = als + ald
            t = jnp.maximum(t, t * 0.2)
            w = jnp.exp(t)
            wtmp[...] = w
            out_rows[i, pl.ds(hb, 16)] = w
            for h in range(nheads):
                out_rows[i, pl.ds(h * 16, 16)] = (
                    rows_s[i, pl.ds(h * 16, 16)] * wtmp[h]
                )

        pltpu.sync_copy(out_rows, acc.at[didx], add=True)

    plsc.subcore_barrier()
    pltpu.sync_copy(acc.at[pl.ds(sid * RPT, RPT)],
                    out_hbm.at[cid, pl.ds(sid * RPT, RPT)])


def _edge_pass(tabs, tabd, src, dst, nheads, ts):
    zrow = jnp.zeros((RPT, ts), jnp.float32)
    kern = pl.kernel(
        functools.partial(_edge_body, nheads, ts),
        out_type=jax.ShapeDtypeStruct((NC, R, ts), jnp.float32),
        mesh=_mesh(),
        scratch_types=[
            pltpu.VMEM((CH,), jnp.int32),
            pltpu.VMEM((CH,), jnp.int32),
            pltpu.VMEM((CH, ts), jnp.float32),
            pltpu.VMEM((CH, TD), jnp.float32),
            pltpu.VMEM((CH, ts), jnp.float32),
            pltpu.VMEM((16,), jnp.float32),
            pltpu.VMEM_SHARED((R, ts), jnp.float32),
        ],
    )
    return kern(tabs, tabd, src, dst, zrow)


_HI = jax.lax.Precision.HIGHEST


def _prep1_body(x_ref, w_ref, as_ref, ad_ref, tabs_ref, tabd_ref):
    h = jnp.dot(x_ref[...], w_ref[...], preferred_element_type=jnp.float32,
                precision=_HI)
    als = jnp.dot(h, as_ref[...], preferred_element_type=jnp.float32,
                  precision=_HI)
    ald = jnp.dot(h, ad_ref[...], preferred_element_type=jnp.float32,
                  precision=_HI)
    br = h.shape[0]
    z8 = jnp.zeros((br, 8), jnp.float32)
    tabs_ref[...] = jnp.concatenate([h, als, z8], axis=1)
    tabd_ref[...] = jnp.concatenate([ald, z8], axis=1)


def _prep1(x_pad, W1, As1, Ad1):
    br = 1280
    grid = (R // br,)
    return pl.pallas_call(
        _prep1_body,
        grid=grid,
        in_specs=[
            pl.BlockSpec((br, D_IN), lambda i: (i, 0)),
            pl.BlockSpec((D_IN, D_IN), lambda i: (0, 0)),
            pl.BlockSpec((D_IN, H1), lambda i: (0, 0)),
            pl.BlockSpec((D_IN, H1), lambda i: (0, 0)),
        ],
        out_specs=[
            pl.BlockSpec((br, TS1), lambda i: (i, 0)),
            pl.BlockSpec((br, TD), lambda i: (i, 0)),
        ],
        out_shape=[
            jax.ShapeDtypeStruct((R, TS1), jnp.float32),
            jax.ShapeDtypeStruct((R, TD), jnp.float32),
        ],
    )(x_pad, W1, As1, Ad1)


def _mid_body(acc_ref, exp8_ref, b1_ref, w2_ref, as2_ref, ad2_ref,
              tabs_ref, tabd_ref):
    a = acc_ref[0] + acc_ref[1]
    hsum = a[:, :D_IN]
    den = a[:, D_IN:D_IN + H1]
    rw = jnp.dot(1.0 / (den + 1e-16), exp8_ref[...],
                 preferred_element_type=jnp.float32, precision=_HI)
    o1 = jax.nn.elu(hsum * rw + b1_ref[...])
    h2 = jnp.dot(o1, w2_ref[...], preferred_element_type=jnp.float32,
                 precision=_HI)
    als = jnp.dot(h2, as2_ref[...], preferred_element_type=jnp.float32,
                  precision=_HI)
    ald = jnp.dot(h2, ad2_ref[...], preferred_element_type=jnp.float32,
                  precision=_HI)
    br = h2.shape[0]
    z15 = jnp.zeros((br, 15), jnp.float32)
    tabs_ref[...] = jnp.concatenate([h2, als, z15], axis=1)
    tabd_ref[...] = jnp.concatenate([ald, z15], axis=1)


def _mid(acc1, Exp8, b1row, W2, As2, Ad2):
    br = 1280
    grid = (R // br,)
    return pl.pallas_call(
        _mid_body,
        grid=grid,
        in_specs=[
            pl.BlockSpec((NC, br, TS1), lambda i: (0, i, 0)),
            pl.BlockSpec((H1, D_IN), lambda i: (0, 0)),
            pl.BlockSpec((1, D_IN), lambda i: (0, 0)),
            pl.BlockSpec((D_IN, OUT), lambda i: (0, 0)),
            pl.BlockSpec((OUT, 1), lambda i: (0, 0)),
            pl.BlockSpec((OUT, 1), lambda i: (0, 0)),
        ],
        out_specs=[
            pl.BlockSpec((br, TS2), lambda i: (i, 0)),
            pl.BlockSpec((br, TD), lambda i: (i, 0)),
        ],
        out_shape=[
            jax.ShapeDtypeStruct((R, TS2), jnp.float32),
            jax.ShapeDtypeStruct((R, TD), jnp.float32),
        ],
    )(acc1, Exp8, b1row, W2, As2, Ad2)


def _final_body(acc_ref, b2_ref, out_ref):
    a = acc_ref[0] + acc_ref[1]
    o = a[:, :OUT] / (a[:, OUT:OUT + 1] + 1e-16) + b2_ref[...]
    m = jnp.max(o, axis=1, keepdims=True)
    l = o - m
    out_ref[...] = l - jnp.log(jnp.sum(jnp.exp(l), axis=1, keepdims=True))


def _final(acc2, b2row):
    br = 2000
    grid = (N // br,)
    return pl.pallas_call(
        _final_body,
        grid=grid,
        in_specs=[
            pl.BlockSpec((NC, br, TS2), lambda i: (0, i, 0)),
            pl.BlockSpec((1, OUT), lambda i: (0, 0)),
        ],
        out_specs=pl.BlockSpec((br, OUT), lambda i: (i, 0)),
        out_shape=jax.ShapeDtypeStruct((N, OUT), jnp.float32),
    )(acc2, b2row)


def kernel(x, edge_index, W1, a_src1, a_dst1, b1, W2, a_src2, a_dst2, b2):
    loop_idx = jnp.arange(N, dtype=jnp.int32)
    padv = jnp.full((EPAD - ETOT,), N, jnp.int32)
    src = jnp.concatenate([edge_index[0].astype(jnp.int32), loop_idx, padv])
    dst = jnp.concatenate([edge_index[1].astype(jnp.int32), loop_idx, padv])

    x_pad = jnp.zeros((R, D_IN), x.dtype).at[:N].set(x)

    lanes = jnp.arange(D_IN)
    heads = jnp.arange(H1)
    mask = (lanes[:, None] // C1) == heads[None, :]
    As1 = jnp.where(mask, a_src1.reshape(-1)[:, None], 0.0)
    Ad1 = jnp.where(mask, a_dst1.reshape(-1)[:, None], 0.0)
    Exp8 = mask.astype(jnp.float32).T
    As2 = a_src2.reshape(OUT, 1)
    Ad2 = a_dst2.reshape(OUT, 1)

    tabs1, tabd1 = _prep1(x_pad, W1, As1, Ad1)
    acc1 = _edge_pass(tabs1, tabd1, src, dst, H1, TS1)
    tabs2, tabd2 = _mid(acc1, Exp8, b1.reshape(1, D_IN), W2, As2, Ad2)
    acc2 = _edge_pass(tabs2, tabd2, src, dst, 1, TS2)
    return _final(acc2, b2.reshape(1, OUT))


# speedup vs baseline: 47.1553x; 47.1553x over previous
import functools

import jax
import jax.numpy as jnp
from jax import lax
from jax.experimental import pallas as pl
from jax.experimental.pallas import tpu as pltpu
from jax.experimental.pallas import tpu_sc as plsc

N = 10000
E = 320000
D_IN = 128
H1, C1 = 8, 16
OUT = 64

NC, NS = 2, 16
NW = NC * NS
R = 10240
RPT = R // NS
ETOT = E + N
CH = 120
EPW = 10320
EPAD = EPW * NW
NCHUNK = EPW // CH

TS1 = 144
TS2 = 80
TD = 16


def _mesh():
    return plsc.VectorSubcoreMesh(
        core_axis_name="c", subcore_axis_name="s", num_cores=NC, num_subcores=NS
    )


def _edge_body(nfeat, nheads, ts, tabs_hbm, tabd_hbm, src_hbm, dst_hbm,
               zrow_hbm, out_hbm, sidx, didx, rows_s, rows_d, out_rows, acc):
    cid = lax.axis_index("c")
    sid = lax.axis_index("s")
    wid = sid * NC + cid
    nq = nfeat // 16
    cph = nq // nheads

    pltpu.sync_copy(zrow_hbm, acc.at[pl.ds(sid * RPT, RPT)])
    plsc.subcore_barrier()

    @pl.loop(0, NCHUNK)
    def _chunks(c):
        off = wid * EPW + c * CH
        pltpu.sync_copy(src_hbm.at[pl.ds(off, CH)], sidx)
        pltpu.sync_copy(dst_hbm.at[pl.ds(off, CH)], didx)
        pltpu.sync_copy(tabs_hbm.at[sidx], rows_s)
        pltpu.sync_copy(tabd_hbm.at[didx], rows_d)

        @pl.loop(0, CH)
        def _edges(i):
            als = rows_s[i, pl.ds(nfeat, 16)]
            ald = rows_d[i, pl.ds(0, 16)]
            t = als + ald
            t = jnp.maximum(t, t * 0.2)
            w = jnp.exp(t)
            out_rows[i, pl.ds(nfeat, 16)] = w
            for q in range(nq):
                out_rows[i, pl.ds(q * 16, 16)] = (
                    rows_s[i, pl.ds(q * 16, 16)] * w[q // cph]
                )

        pltpu.sync_copy(out_rows, acc.at[didx], add=True)

    plsc.subcore_barrier()
    pltpu.sync_copy(acc.at[pl.ds(sid * RPT, RPT)],
                    out_hbm.at[cid, pl.ds(sid * RPT, RPT)])


def _edge_pass(tabs, tabd, src, dst, nfeat, nheads, ts):
    zrow = jnp.zeros((RPT, ts), jnp.float32)
    kern = pl.kernel(
        functools.partial(_edge_body, nfeat, nheads, ts),
        out_type=jax.ShapeDtypeStruct((NC, R, ts), jnp.float32),
        mesh=_mesh(),
        compiler_params=pltpu.CompilerParams(use_tc_tiling_on_sc=False),
        scratch_types=[
            pltpu.VMEM((CH,), jnp.int32),
            pltpu.VMEM((CH,), jnp.int32),
            pltpu.VMEM((CH, ts), jnp.float32),
            pltpu.VMEM((CH, TD), jnp.float32),
            pltpu.VMEM((CH, ts), jnp.float32),
            pltpu.VMEM_SHARED((R, ts), jnp.float32),
        ],
    )
    return kern(tabs, tabd, src, dst, zrow)


_HI = jax.lax.Precision.HIGHEST


def _prep1_body(x_ref, w_ref, as_ref, ad_ref, tabs_ref, tabd_ref):
    h = jnp.dot(x_ref[...], w_ref[...], preferred_element_type=jnp.float32,
                precision=_HI)
    als = jnp.dot(h, as_ref[...], preferred_element_type=jnp.float32,
                  precision=_HI)
    ald = jnp.dot(h, ad_ref[...], preferred_element_type=jnp.float32,
                  precision=_HI)
    br = h.shape[0]
    z8 = jnp.zeros((br, 8), jnp.float32)
    tabs_ref[...] = jnp.concatenate([h, als, z8], axis=1)
    tabd_ref[...] = jnp.concatenate([ald, z8], axis=1)


def _prep1(x_pad, W1, As1, Ad1):
    br = 1280
    grid = (R // br,)
    return pl.pallas_call(
        _prep1_body,
        grid=grid,
        in_specs=[
            pl.BlockSpec((br, D_IN), lambda i: (i, 0)),
            pl.BlockSpec((D_IN, D_IN), lambda i: (0, 0)),
            pl.BlockSpec((D_IN, H1), lambda i: (0, 0)),
            pl.BlockSpec((D_IN, H1), lambda i: (0, 0)),
        ],
        out_specs=[
            pl.BlockSpec((br, TS1), lambda i: (i, 0)),
            pl.BlockSpec((br, TD), lambda i: (i, 0)),
        ],
        out_shape=[
            jax.ShapeDtypeStruct((R, TS1), jnp.float32),
            jax.ShapeDtypeStruct((R, TD), jnp.float32),
        ],
    )(x_pad, W1, As1, Ad1)


def _mid_body(acc_ref, exp8_ref, b1_ref, w2_ref, as2_ref, ad2_ref,
              tabs_ref, tabd_ref):
    a = acc_ref[0] + acc_ref[1]
    hsum = a[:, :D_IN]
    den = a[:, D_IN:D_IN + H1]
    rw = jnp.dot(1.0 / (den + 1e-16), exp8_ref[...],
                 preferred_element_type=jnp.float32, precision=_HI)
    p = hsum * rw + b1_ref[...]
    o1 = jnp.where(p > 0, p, jnp.exp(jnp.minimum(p, 0.0)) - 1.0)
    h2 = jnp.dot(o1, w2_ref[...], preferred_element_type=jnp.float32,
                 precision=_HI)
    als = jnp.dot(h2, as2_ref[...], preferred_element_type=jnp.float32,
                  precision=_HI)
    ald = jnp.dot(h2, ad2_ref[...], preferred_element_type=jnp.float32,
                  precision=_HI)
    br = h2.shape[0]
    z15 = jnp.zeros((br, 15), jnp.float32)
    tabs_ref[...] = jnp.concatenate([h2, als, z15], axis=1)
    tabd_ref[...] = jnp.concatenate([ald, z15], axis=1)


def _mid(acc1, Exp8, b1row, W2, As2, Ad2):
    br = 1280
    grid = (R // br,)
    return pl.pallas_call(
        _mid_body,
        grid=grid,
        in_specs=[
            pl.BlockSpec((NC, br, TS1), lambda i: (0, i, 0)),
            pl.BlockSpec((H1, D_IN), lambda i: (0, 0)),
            pl.BlockSpec((1, D_IN), lambda i: (0, 0)),
            pl.BlockSpec((D_IN, OUT), lambda i: (0, 0)),
            pl.BlockSpec((OUT, 1), lambda i: (0, 0)),
            pl.BlockSpec((OUT, 1), lambda i: (0, 0)),
        ],
        out_specs=[
            pl.BlockSpec((br, TS2), lambda i: (i, 0)),
            pl.BlockSpec((br, TD), lambda i: (i, 0)),
        ],
        out_shape=[
            jax.ShapeDtypeStruct((R, TS2), jnp.float32),
            jax.ShapeDtypeStruct((R, TD), jnp.float32),
        ],
    )(acc1, Exp8, b1row, W2, As2, Ad2)


def _final_body(acc_ref, b2_ref, out_ref):
    a = acc_ref[0] + acc_ref[1]
    o = a[:, :OUT] / (a[:, OUT:OUT + 1] + 1e-16) + b2_ref[...]
    m = jnp.max(o, axis=1, keepdims=True)
    l = o - m
    out_ref[...] = l - jnp.log(jnp.sum(jnp.exp(l), axis=1, keepdims=True))


def _final(acc2, b2row):
    br = 2000
    grid = (N // br,)
    return pl.pallas_call(
        _final_body,
        grid=grid,
        in_specs=[
            pl.BlockSpec((NC, br, TS2), lambda i: (0, i, 0)),
            pl.BlockSpec((1, OUT), lambda i: (0, 0)),
        ],
        out_specs=pl.BlockSpec((br, OUT), lambda i: (i, 0)),
        out_shape=jax.ShapeDtypeStruct((N, OUT), jnp.float32),
    )(acc2, b2row)


def kernel(x, edge_index, W1, a_src1, a_dst1, b1, W2, a_src2, a_dst2, b2):
    loop_idx = jnp.arange(N, dtype=jnp.int32)
    padv = jnp.full((EPAD - ETOT,), N, jnp.int32)
    src = jnp.concatenate([edge_index[0].astype(jnp.int32), loop_idx, padv])
    dst = jnp.concatenate([edge_index[1].astype(jnp.int32), loop_idx, padv])

    x_pad = jnp.zeros((R, D_IN), x.dtype).at[:N].set(x)

    lanes = jnp.arange(D_IN)
    heads = jnp.arange(H1)
    mask = (lanes[:, None] // C1) == heads[None, :]
    As1 = jnp.where(mask, a_src1.reshape(-1)[:, None], 0.0)
    Ad1 = jnp.where(mask, a_dst1.reshape(-1)[:, None], 0.0)
    Exp8 = mask.astype(jnp.float32).T
    As2 = a_src2.reshape(OUT, 1)
    Ad2 = a_dst2.reshape(OUT, 1)

    tabs1, tabd1 = _prep1(x_pad, W1, As1, Ad1)
    acc1 = _edge_pass(tabs1, tabd1, src, dst, D_IN, H1, TS1)
    tabs2, tabd2 = _mid(acc1, Exp8, b1.reshape(1, D_IN), W2, As2, Ad2)
    acc2 = _edge_pass(tabs2, tabd2, src, dst, OUT, 1, TS2)
    return _final(acc2, b2.reshape(1, OUT))
